# baseline (device time: 23317 ns/iter reference)
import jax
import jax.numpy as jnp
from jax import lax
from jax.experimental import pallas as pl
from jax.experimental.pallas import tpu as pltpu

N_Z = 4
PAGES_PER_SHARD = 64
BS = 16
H = 8
D = 64
B = 8
ROWS = PAGES_PER_SHARD * BS
NC = B * H
SROWS = NC + 2


def _body(k_ref, v_ref, q_ref, w_ref, out_ref, comm_ref, send_sems, recv_sems):
    me = lax.axis_index("z")
    mx = lax.axis_index("x")
    my = lax.axis_index("y")

    barrier_sem = pltpu.get_barrier_semaphore()
    for d in range(1, N_Z):
        pl.semaphore_signal(
            barrier_sem, inc=1,
            device_id=(mx, my, (me + d) % N_Z),
            device_id_type=pl.DeviceIdType.MESH,
        )
    pl.semaphore_wait(barrier_sem, N_Z - 1)

    for h in range(H):
        Kh = k_ref[:, :, h, :].reshape(ROWS, D)
        Qh = q_ref[h]
        Sh = lax.dot_general(
            Kh, Qh,
            dimension_numbers=(((1,), (1,)), ((), ())),
            preferred_element_type=jnp.float32,
        ) * (D ** -0.5)
        Wh = w_ref[:, h * B:(h + 1) * B]
        Smh = jnp.where(Wh > 0, Sh, -1e30)
        mh = jnp.max(Smh, axis=0)
        Eh = Wh * jnp.exp(Smh - mh[None, :])
        lh = jnp.sum(Eh, axis=0)
        Vh = v_ref[:, :, h, :].reshape(ROWS, D)
        Oh = lax.dot_general(
            Eh, Vh,
            dimension_numbers=(((0,), (0,)), ((), ())),
            preferred_element_type=jnp.float32,
        )
        comm_ref[pl.ds(me, 1), h * B:(h + 1) * B, :] = Oh[None]
        comm_ref[pl.ds(me, 1), NC:NC + 1, h * B:(h + 1) * B] = mh.reshape(1, 1, B)
        comm_ref[pl.ds(me, 1), NC + 1:NC + 2, h * B:(h + 1) * B] = lh.reshape(1, 1, B)

    rdmas = []
    for d in range(1, N_Z):
        rdma = pltpu.make_async_remote_copy(
            src_ref=comm_ref.at[pl.ds(me, 1)],
            dst_ref=comm_ref.at[pl.ds(me, 1)],
            send_sem=send_sems.at[d - 1],
            recv_sem=recv_sems.at[d - 1],
            device_id=(mx, my, (me + d) % N_Z),
            device_id_type=pl.DeviceIdType.MESH,
        )
        rdma.start()
        rdmas.append(rdma)
    for rdma in rdmas:
        rdma.wait()

    ms = [comm_ref[p, NC, :] for p in range(N_Z)]
    Mx = jnp.maximum(jnp.maximum(ms[0], ms[1]), jnp.maximum(ms[2], ms[3]))
    sc = [jnp.exp(ms[p] - Mx) for p in range(N_Z)]
    L = sum(sc[p] * comm_ref[p, NC + 1, :] for p in range(N_Z))
    Ofin = sum(comm_ref[p, 0:NC, :] * sc[p][:, None] for p in range(N_Z))
    Ofin = Ofin / L[:, None]

    for h in range(H):
        out_ref[:, 0, h, :] = Ofin[h * B:(h + 1) * B, :]


def kernel(Q, K, V, bt, lens):
    z = lax.axis_index("z")
    pages_local = z * PAGES_PER_SHARD + jnp.arange(PAGES_PER_SHARD)
    valid = jnp.arange(64)[None, :] < lens[:, None]
    cnt = jnp.sum(
        (bt[:, :, None] == pages_local[None, None, :]) & valid[:, :, None],
        axis=1,
    ).astype(jnp.float32)
    Wrow = jnp.repeat(cnt, BS, axis=1)
    W = jnp.tile(Wrow.T, (1, H))

    Qhb = Q[:, 0, :, :].transpose(1, 0, 2)

    return pl.pallas_call(
        _body,
        out_shape=jax.ShapeDtypeStruct((B, 1, H, D), jnp.float32),
        in_specs=[pl.BlockSpec(memory_space=pltpu.VMEM)] * 4,
        out_specs=pl.BlockSpec(memory_space=pltpu.VMEM),
        scratch_shapes=[
            pltpu.VMEM((N_Z, SROWS, NC), jnp.float32),
            pltpu.SemaphoreType.DMA((N_Z - 1,)),
            pltpu.SemaphoreType.DMA((N_Z - 1,)),
        ],
        compiler_params=pltpu.CompilerParams(collective_id=0),
    )(K, V, Qhb, W)
